# baseline (speedup 1.0000x reference)
import functools

import jax
import jax.numpy as jnp
from jax import lax
from jax.experimental import pallas as pl
from jax.experimental.pallas import tpu as pltpu
from jax.experimental.pallas import tpu_sc as plsc

_MU = 0.02
_NBINS = 10
_ALPHA = 0.75
_N = 8388608
_COLS = 128
_ROWS = _N // _COLS

_SC_ROWS = 1024
_TC_ROWS = _ROWS - _SC_ROWS

_BLK = 1024
_CH = 8
_UNROLL = 16
_NCH = _BLK // (_CH * _UNROLL)

_NW = 32
_L = 16
_SC_N = _SC_ROWS * _COLS
_PER_W = _SC_N // _NW
_SC_CHUNK = min(16384, _PER_W)
_SC_NCHUNK = max(1, _PER_W // _SC_CHUNK)
_SC_U = 16


def _tc_body(p_ref, t_ref, out_ref, acc_ref):
    step = pl.program_id(0)

    @pl.when(step == 0)
    def _init():
        for k in range(2 * _NBINS + 1):
            acc_ref[k] = jnp.float32(0.0)

    zero = jnp.zeros((_CH, _COLS), jnp.float32)

    def chunk(i, carry):
        cnt, s, ov = carry
        cnt = list(cnt)
        s = list(s)
        for u in range(_UNROLL):
            base = (i * _UNROLL + u) * _CH
            p = p_ref[pl.ds(base, _CH), :]
            t = t_ref[pl.ds(base, _CH), :]
            d = jnp.abs(p - t)
            loss = jnp.where(d < _MU, (0.5 / _MU) * d * d, d - 0.5 * _MU)
            g = jnp.abs(jnp.tanh(p) - jnp.tanh(t))
            bf = jnp.trunc(g * _NBINS)
            for k in range(_NBINS - 1):
                m = bf == jnp.float32(k)
                cnt[k] = cnt[k] + jnp.where(m, 1.0, 0.0)
                s[k] = s[k] + jnp.where(m, loss, 0.0)
            m9 = bf >= jnp.float32(_NBINS - 1)
            cnt[9] = cnt[9] + jnp.where(m9, 1.0, 0.0)
            s[9] = s[9] + jnp.where(m9, loss, 0.0)
            ov = ov + jnp.where(bf >= jnp.float32(_NBINS), 1.0, 0.0)
        return tuple(cnt), tuple(s), ov

    init = (tuple(zero for _ in range(_NBINS)),
            tuple(zero for _ in range(_NBINS)), zero)
    cnt, s, ov = lax.fori_loop(0, _NCH, chunk, init)

    for k in range(_NBINS):
        acc_ref[k] += jnp.sum(cnt[k])
        acc_ref[_NBINS + k] += jnp.sum(s[k])
    acc_ref[2 * _NBINS] += jnp.sum(ov)

    @pl.when(step == pl.num_programs(0) - 1)
    def _finish():
        for k in range(2 * _NBINS + 1):
            out_ref[k] = acc_ref[k]


def _tc_partials(p2, t2):
    grid = _TC_ROWS // _BLK
    return pl.pallas_call(
        _tc_body,
        grid=(grid,),
        in_specs=[
            pl.BlockSpec((_BLK, _COLS), lambda i: (i, 0)),
            pl.BlockSpec((_BLK, _COLS), lambda i: (i, 0)),
        ],
        out_specs=pl.BlockSpec(memory_space=pltpu.SMEM),
        out_shape=jax.ShapeDtypeStruct((2 * _NBINS + 1,), jnp.float32),
        scratch_shapes=[pltpu.SMEM((2 * _NBINS + 1,), jnp.float32)],
    )(p2, t2)


def _sc_tec_body(p_hbm, t_hbm, out_hbm, pb0, tb0, pb1, tb1, acc, sem0, sem1):
    c = lax.axis_index("c")
    s = lax.axis_index("s")
    wid = s * 2 + c
    base = wid * _PER_W

    for r in range(2 * _NBINS):
        acc[pl.ds(r * _L, _L)] = jnp.zeros((_L,), jnp.float32)
    lane = lax.iota(jnp.int32, _L)
    ones = jnp.ones((_L,), jnp.float32)

    pbufs = (pb0, pb1)
    tbufs = (tb0, tb1)
    sems = (sem0, sem1)

    def issue(ci, b):
        off = base + ci * _SC_CHUNK
        hp = pltpu.async_copy(p_hbm.at[pl.ds(off, _SC_CHUNK)], pbufs[b], sems[b])
        ht = pltpu.async_copy(t_hbm.at[pl.ds(off, _SC_CHUNK)], tbufs[b], sems[b])
        return hp, ht

    def process(pb, tb):
        def it(j, carry):
            for u in range(_SC_U):
                off = (j * _SC_U + u) * _L
                p = pb[pl.ds(off, _L)]
                t = tb[pl.ds(off, _L)]
                d = jnp.abs(p - t)
                loss = jnp.where(d < _MU, (0.5 / _MU) * d * d, d - 0.5 * _MU)
                u = jnp.exp(jnp.minimum(p + p, 60.0))
                v = jnp.exp(jnp.minimum(t + t, 60.0))
                gs = jnp.abs((2.0 * _NBINS) * (u - v)) / ((u + 1.0) * (v + 1.0))
                bf = gs.astype(jnp.int32)
                bl = jnp.minimum(bf, _NBINS - 1)
                flat = bl * _L + lane
                plsc.addupdate_scatter(acc, [flat + _NBINS * _L], loss)
                plsc.addupdate_scatter(acc, [flat], ones,
                                       mask=bf < _NBINS)
            return carry
        lax.fori_loop(0, _SC_CHUNK // (_L * _SC_U), it, 0)

    h = issue(0, 0)
    for ci in range(_SC_NCHUNK):
        b = ci % 2
        hp, ht = h
        hp.wait()
        ht.wait()
        if ci + 1 < _SC_NCHUNK:
            h = issue(ci + 1, (ci + 1) % 2)
        process(pbufs[b], tbufs[b])

    pltpu.sync_copy(acc, out_hbm.at[wid])


def _sc_partials(p_flat, t_flat):
    mesh = plsc.VectorSubcoreMesh(core_axis_name="c", subcore_axis_name="s")
    k = functools.partial(
        pl.kernel,
        mesh=mesh,
        out_type=jax.ShapeDtypeStruct((_NW, 2 * _NBINS * _L), jnp.float32),
        scratch_types=[
            pltpu.VMEM((_SC_CHUNK,), jnp.float32),
            pltpu.VMEM((_SC_CHUNK,), jnp.float32),
            pltpu.VMEM((_SC_CHUNK,), jnp.float32),
            pltpu.VMEM((_SC_CHUNK,), jnp.float32),
            pltpu.VMEM((2 * _NBINS * _L,), jnp.float32),
            pltpu.SemaphoreType.DMA,
            pltpu.SemaphoreType.DMA,
        ],
        compiler_params=pltpu.CompilerParams(needs_layout_passes=False),
    )(_sc_tec_body)
    return k(p_flat, t_flat)


def kernel(pred, target):
    cnt = jnp.zeros((_NBINS,), jnp.float32)
    ssum = jnp.zeros((_NBINS,), jnp.float32)

    if _SC_ROWS > 0:
        sc_out = _sc_partials(pred[_N - _SC_N:], target[_N - _SC_N:])

    if _TC_ROWS > 0:
        ntc = _TC_ROWS * _COLS
        tc = _tc_partials(pred[:ntc].reshape(_TC_ROWS, _COLS),
                          target[:ntc].reshape(_TC_ROWS, _COLS))
        tc_cnt = tc[:_NBINS]
        tc_cnt = tc_cnt.at[_NBINS - 1].add(-tc[2 * _NBINS])
        cnt = cnt + tc_cnt
        ssum = ssum + tc[_NBINS:2 * _NBINS]

    if _SC_ROWS > 0:
        sc = sc_out
        sc = sc.reshape(_NW, 2 * _NBINS, _L)
        cnt = cnt + jnp.sum(sc[:, :_NBINS, :], axis=(0, 2))
        ssum = ssum + jnp.sum(sc[:, _NBINS:, :], axis=(0, 2))

    w = jnp.maximum(cnt, 1.0) ** (-_ALPHA)
    return jnp.sum(w * ssum) / _N


# speedup vs baseline: 1.8531x; 1.8531x over previous
import jax
import jax.numpy as jnp
from jax import lax
from jax.experimental import pallas as pl
from jax.experimental.pallas import tpu as pltpu

_MU = 0.02
_NBINS = 10
_ALPHA = 0.75
_N = 8388608
_COLS = 128
_ROWS = _N // _COLS
_BLK = 4096
_GRID = _ROWS // _BLK
_CH = 8
_UNROLL = 16
_NCH = _BLK // (_CH * _UNROLL)


def _ghmr_body(p_ref, t_ref, out_ref, acc_ref):
    step = pl.program_id(0)

    @pl.when(step == 0)
    def _init():
        for k in range(2 * _NBINS + 1):
            acc_ref[k] = jnp.float32(0.0)

    zero = jnp.zeros((_CH, _COLS), jnp.float32)

    def chunk(i, carry):
        cnt, s, ov = carry
        cnt = list(cnt)
        s = list(s)
        for u in range(_UNROLL):
            base = (i * _UNROLL + u) * _CH
            p = p_ref[pl.ds(base, _CH), :]
            t = t_ref[pl.ds(base, _CH), :]
            d = jnp.abs(p - t)
            loss = jnp.where(d < _MU, (0.5 / _MU) * d * d, d - 0.5 * _MU)
            g = jnp.abs(jnp.tanh(p) - jnp.tanh(t))
            bf = jnp.trunc(g * _NBINS)
            for k in range(_NBINS - 1):
                m = bf == jnp.float32(k)
                cnt[k] = cnt[k] + jnp.where(m, 1.0, 0.0)
                s[k] = s[k] + jnp.where(m, loss, 0.0)
            m9 = bf >= jnp.float32(_NBINS - 1)
            cnt[9] = cnt[9] + jnp.where(m9, 1.0, 0.0)
            s[9] = s[9] + jnp.where(m9, loss, 0.0)
            ov = ov + jnp.where(bf >= jnp.float32(_NBINS), 1.0, 0.0)
        return tuple(cnt), tuple(s), ov

    init = (tuple(zero for _ in range(_NBINS)),
            tuple(zero for _ in range(_NBINS)), zero)
    cnt, s, ov = lax.fori_loop(0, _NCH, chunk, init)

    for k in range(_NBINS):
        acc_ref[k] += jnp.sum(cnt[k])
        acc_ref[_NBINS + k] += jnp.sum(s[k])
    acc_ref[2 * _NBINS] += jnp.sum(ov)

    @pl.when(step == _GRID - 1)
    def _finish():
        total = jnp.float32(0.0)
        for k in range(_NBINS):
            c = acc_ref[k]
            if k == _NBINS - 1:
                c = c - acc_ref[2 * _NBINS]
            c = jnp.maximum(c, 1.0)
            w = jnp.exp(-_ALPHA * jnp.log(c))
            total = total + w * acc_ref[_NBINS + k]
        out_ref[0] = total / _N


def kernel(pred, target):
    p2 = pred.reshape(_ROWS, _COLS)
    t2 = target.reshape(_ROWS, _COLS)
    out = pl.pallas_call(
        _ghmr_body,
        grid=(_GRID,),
        in_specs=[
            pl.BlockSpec((_BLK, _COLS), lambda i: (i, 0)),
            pl.BlockSpec((_BLK, _COLS), lambda i: (i, 0)),
        ],
        out_specs=pl.BlockSpec(memory_space=pltpu.SMEM),
        out_shape=jax.ShapeDtypeStruct((1,), jnp.float32),
        scratch_shapes=[pltpu.SMEM((2 * _NBINS + 1,), jnp.float32)],
    )(p2, t2)
    return out[0]


# speedup vs baseline: 1.8920x; 1.0210x over previous
import jax
import jax.numpy as jnp
from jax import lax
from jax.experimental import pallas as pl
from jax.experimental.pallas import tpu as pltpu

_MU = 0.02
_NBINS = 10
_ALPHA = 0.75
_N = 8388608
_COLS = 128
_ROWS = _N // _COLS
_BLK = 4096
_GRID = _ROWS // _BLK
_CH = 8
_UNROLL = 32
_NCH = _BLK // (_CH * _UNROLL)


def _ghmr_body(p_ref, t_ref, out_ref, acc_ref):
    step = pl.program_id(0)

    @pl.when(step == 0)
    def _init():
        for k in range(2 * _NBINS + 1):
            acc_ref[k] = jnp.float32(0.0)

    zero = jnp.zeros((_CH, _COLS), jnp.float32)

    def chunk(i, carry):
        cnt, s, ov = carry
        cnt = list(cnt)
        s = list(s)
        for u in range(_UNROLL):
            base = (i * _UNROLL + u) * _CH
            p = p_ref[pl.ds(base, _CH), :]
            t = t_ref[pl.ds(base, _CH), :]
            d = jnp.abs(p - t)
            loss = jnp.where(d < _MU, (0.5 / _MU) * d * d, d - 0.5 * _MU)
            g = jnp.abs(jnp.tanh(p) - jnp.tanh(t))
            bf = jnp.trunc(g * _NBINS)
            for k in range(_NBINS - 1):
                m = bf == jnp.float32(k)
                cnt[k] = cnt[k] + jnp.where(m, 1.0, 0.0)
                s[k] = s[k] + jnp.where(m, loss, 0.0)
            m9 = bf >= jnp.float32(_NBINS - 1)
            cnt[9] = cnt[9] + jnp.where(m9, 1.0, 0.0)
            s[9] = s[9] + jnp.where(m9, loss, 0.0)
            ov = ov + jnp.where(bf >= jnp.float32(_NBINS), 1.0, 0.0)
        return tuple(cnt), tuple(s), ov

    init = (tuple(zero for _ in range(_NBINS)),
            tuple(zero for _ in range(_NBINS)), zero)
    cnt, s, ov = lax.fori_loop(0, _NCH, chunk, init)

    for k in range(_NBINS):
        acc_ref[k] += jnp.sum(cnt[k])
        acc_ref[_NBINS + k] += jnp.sum(s[k])
    acc_ref[2 * _NBINS] += jnp.sum(ov)

    @pl.when(step == _GRID - 1)
    def _finish():
        total = jnp.float32(0.0)
        for k in range(_NBINS):
            c = acc_ref[k]
            if k == _NBINS - 1:
                c = c - acc_ref[2 * _NBINS]
            c = jnp.maximum(c, 1.0)
            w = jnp.exp(-_ALPHA * jnp.log(c))
            total = total + w * acc_ref[_NBINS + k]
        out_ref[0] = total / _N


def kernel(pred, target):
    p2 = pred.reshape(_ROWS, _COLS)
    t2 = target.reshape(_ROWS, _COLS)
    out = pl.pallas_call(
        _ghmr_body,
        grid=(_GRID,),
        in_specs=[
            pl.BlockSpec((_BLK, _COLS), lambda i: (i, 0)),
            pl.BlockSpec((_BLK, _COLS), lambda i: (i, 0)),
        ],
        out_specs=pl.BlockSpec(memory_space=pltpu.SMEM),
        out_shape=jax.ShapeDtypeStruct((1,), jnp.float32),
        scratch_shapes=[pltpu.SMEM((2 * _NBINS + 1,), jnp.float32)],
    )(p2, t2)
    return out[0]


# speedup vs baseline: 1.9127x; 1.0109x over previous
import jax
import jax.numpy as jnp
from jax import lax
from jax.experimental import pallas as pl
from jax.experimental.pallas import tpu as pltpu

_MU = 0.02
_NBINS = 10
_ALPHA = 0.75
_N = 8388608
_COLS = 128
_ROWS = _N // _COLS
_BLK = 4096
_GRID = _ROWS // _BLK
_CH = 8
_UNROLL = 64
_NCH = _BLK // (_CH * _UNROLL)


def _ghmr_body(p_ref, t_ref, out_ref, acc_ref):
    step = pl.program_id(0)

    @pl.when(step == 0)
    def _init():
        for k in range(2 * _NBINS + 1):
            acc_ref[k] = jnp.float32(0.0)

    zero = jnp.zeros((_CH, _COLS), jnp.float32)

    def chunk(i, carry):
        cnt, s, ov = carry
        cnt = list(cnt)
        s = list(s)
        for u in range(_UNROLL):
            base = (i * _UNROLL + u) * _CH
            p = p_ref[pl.ds(base, _CH), :]
            t = t_ref[pl.ds(base, _CH), :]
            d = jnp.abs(p - t)
            loss = jnp.where(d < _MU, (0.5 / _MU) * d * d, d - 0.5 * _MU)
            g = jnp.abs(jnp.tanh(p) - jnp.tanh(t))
            bf = jnp.trunc(g * _NBINS)
            for k in range(_NBINS - 1):
                m = bf == jnp.float32(k)
                cnt[k] = cnt[k] + jnp.where(m, 1.0, 0.0)
                s[k] = s[k] + jnp.where(m, loss, 0.0)
            m9 = bf >= jnp.float32(_NBINS - 1)
            cnt[9] = cnt[9] + jnp.where(m9, 1.0, 0.0)
            s[9] = s[9] + jnp.where(m9, loss, 0.0)
            ov = ov + jnp.where(bf >= jnp.float32(_NBINS), 1.0, 0.0)
        return tuple(cnt), tuple(s), ov

    init = (tuple(zero for _ in range(_NBINS)),
            tuple(zero for _ in range(_NBINS)), zero)
    cnt, s, ov = lax.fori_loop(0, _NCH, chunk, init)

    for k in range(_NBINS):
        acc_ref[k] += jnp.sum(cnt[k])
        acc_ref[_NBINS + k] += jnp.sum(s[k])
    acc_ref[2 * _NBINS] += jnp.sum(ov)

    @pl.when(step == _GRID - 1)
    def _finish():
        total = jnp.float32(0.0)
        for k in range(_NBINS):
            c = acc_ref[k]
            if k == _NBINS - 1:
                c = c - acc_ref[2 * _NBINS]
            c = jnp.maximum(c, 1.0)
            w = jnp.exp(-_ALPHA * jnp.log(c))
            total = total + w * acc_ref[_NBINS + k]
        out_ref[0] = total / _N


def kernel(pred, target):
    p2 = pred.reshape(_ROWS, _COLS)
    t2 = target.reshape(_ROWS, _COLS)
    out = pl.pallas_call(
        _ghmr_body,
        grid=(_GRID,),
        in_specs=[
            pl.BlockSpec((_BLK, _COLS), lambda i: (i, 0)),
            pl.BlockSpec((_BLK, _COLS), lambda i: (i, 0)),
        ],
        out_specs=pl.BlockSpec(memory_space=pltpu.SMEM),
        out_shape=jax.ShapeDtypeStruct((1,), jnp.float32),
        scratch_shapes=[pltpu.SMEM((2 * _NBINS + 1,), jnp.float32)],
    )(p2, t2)
    return out[0]
